# baseline (speedup 1.0000x reference)
import jax
import jax.numpy as jnp
from jax.experimental import pallas as pl
from jax.experimental.pallas import tpu as pltpu

_N = 10000
_D_IN = 128
_D_OUT = 128
_BM = 80


def _gcn_kernel(x_ref, w_ref, b_ref, a_ref, out_ref, temple_ref):
    @pl.when(pl.program_id(0) == 0)
    def _compute_temple():
        temple_ref[...] = jnp.dot(
            x_ref[...], w_ref[...],
            preferred_element_type=jnp.float32,
            precision=jax.lax.Precision.HIGHEST,
        )

    out_ref[...] = (
        jnp.dot(
            a_ref[...], temple_ref[...],
            preferred_element_type=jnp.float32,
            precision=jax.lax.Precision.DEFAULT,
        )
        + b_ref[...]
    )


def kernel(adjacency, features_matrix, weights, bias):
    bias2d = bias.reshape(1, _D_OUT)
    return pl.pallas_call(
        _gcn_kernel,
        grid=(_N // _BM,),
        in_specs=[
            pl.BlockSpec((_N, _D_IN), lambda i: (0, 0)),
            pl.BlockSpec((_D_IN, _D_OUT), lambda i: (0, 0)),
            pl.BlockSpec((1, _D_OUT), lambda i: (0, 0)),
            pl.BlockSpec((_BM, _N), lambda i: (i, 0)),
        ],
        out_specs=pl.BlockSpec((_BM, _D_OUT), lambda i: (i, 0)),
        out_shape=jax.ShapeDtypeStruct((_N, _D_OUT), jnp.float32),
        scratch_shapes=[pltpu.VMEM((_N, _D_IN), jnp.float32)],
    )(features_matrix, weights, bias2d, adjacency)


# speedup vs baseline: 1.3311x; 1.3311x over previous
import jax
import jax.numpy as jnp
from jax.experimental import pallas as pl

_N = 10000
_D_IN = 128
_D_OUT = 128
_BM = 400


def _temple_kernel(x_ref, w_ref, t_ref):
    t_ref[...] = jnp.dot(
        x_ref[...], w_ref[...],
        preferred_element_type=jnp.float32,
        precision=jax.lax.Precision.DEFAULT,
    )


def _spmm_kernel(t_ref, b_ref, a_ref, out_ref):
    out_ref[...] = (
        jnp.dot(
            a_ref[...], t_ref[...],
            preferred_element_type=jnp.float32,
            precision=jax.lax.Precision.DEFAULT,
        )
        + b_ref[...]
    )


def kernel(adjacency, features_matrix, weights, bias):
    temple = pl.pallas_call(
        _temple_kernel,
        out_shape=jax.ShapeDtypeStruct((_N, _D_IN), jnp.float32),
    )(features_matrix, weights)

    bias2d = bias.reshape(1, _D_OUT)
    return pl.pallas_call(
        _spmm_kernel,
        grid=(_N // _BM,),
        in_specs=[
            pl.BlockSpec((_N, _D_OUT), lambda i: (0, 0)),
            pl.BlockSpec((1, _D_OUT), lambda i: (0, 0)),
            pl.BlockSpec((_BM, _N), lambda i: (i, 0)),
        ],
        out_specs=pl.BlockSpec((_BM, _D_OUT), lambda i: (i, 0)),
        out_shape=jax.ShapeDtypeStruct((_N, _D_OUT), jnp.float32),
    )(temple, bias2d, adjacency)


# speedup vs baseline: 1.3618x; 1.0230x over previous
import jax
import jax.numpy as jnp
from jax.experimental import pallas as pl
from jax.experimental.pallas import tpu as pltpu

_N = 10000
_D_IN = 128
_D_OUT = 128
_BM = 200


def _gcn_kernel(x_ref, w_ref, b_ref, a0_ref, a1_ref, out_ref, temple_ref):
    @pl.when(pl.program_id(0) == 0)
    def _compute_temple():
        temple_ref[...] = jnp.dot(
            x_ref[...], w_ref[...],
            preferred_element_type=jnp.float32,
            precision=jax.lax.Precision.DEFAULT,
        )

    out_ref[0:_BM, :] = (
        jnp.dot(
            a0_ref[...], temple_ref[...],
            preferred_element_type=jnp.float32,
            precision=jax.lax.Precision.DEFAULT,
        )
        + b_ref[...]
    )
    out_ref[_BM : 2 * _BM, :] = (
        jnp.dot(
            a1_ref[...], temple_ref[...],
            preferred_element_type=jnp.float32,
            precision=jax.lax.Precision.DEFAULT,
        )
        + b_ref[...]
    )


def kernel(adjacency, features_matrix, weights, bias):
    bias2d = bias.reshape(1, _D_OUT)
    return pl.pallas_call(
        _gcn_kernel,
        grid=(_N // (2 * _BM),),
        in_specs=[
            pl.BlockSpec((_N, _D_IN), lambda i: (0, 0)),
            pl.BlockSpec((_D_IN, _D_OUT), lambda i: (0, 0)),
            pl.BlockSpec((1, _D_OUT), lambda i: (0, 0)),
            pl.BlockSpec((_BM, _N), lambda i: (2 * i, 0)),
            pl.BlockSpec((_BM, _N), lambda i: (2 * i + 1, 0)),
        ],
        out_specs=pl.BlockSpec((2 * _BM, _D_OUT), lambda i: (i, 0)),
        out_shape=jax.ShapeDtypeStruct((_N, _D_OUT), jnp.float32),
        scratch_shapes=[pltpu.VMEM((_N, _D_IN), jnp.float32)],
    )(features_matrix, weights, bias2d, adjacency, adjacency)


# speedup vs baseline: 1.3978x; 1.0265x over previous
import jax
import jax.numpy as jnp
from jax.experimental import pallas as pl
from jax.experimental.pallas import tpu as pltpu

_N = 10000
_D_IN = 128
_D_OUT = 128
_BM = 200
_NBLK = _N // _BM
_DEPTH = 3


def _a_copy(a_hbm, a_buf, sems, block, slot):
    return pltpu.make_async_copy(
        a_hbm.at[pl.ds(block * _BM, _BM), :], a_buf.at[slot], sems.at[slot]
    )


def _gcn_kernel(x_ref, w_ref, b_ref, a_hbm, out_ref, temple_ref, a_buf, sems):
    for k in range(_DEPTH):
        _a_copy(a_hbm, a_buf, sems, k, k).start()

    temple_ref[...] = jnp.dot(
        x_ref[...], w_ref[...],
        preferred_element_type=jnp.float32,
        precision=jax.lax.Precision.DEFAULT,
    )

    def body(i, carry):
        slot = jax.lax.rem(i, _DEPTH)
        _a_copy(a_hbm, a_buf, sems, i, slot).wait()
        out_ref[pl.ds(i * _BM, _BM), :] = (
            jnp.dot(
                a_buf[slot], temple_ref[...],
                preferred_element_type=jnp.float32,
                precision=jax.lax.Precision.DEFAULT,
            )
            + b_ref[...]
        )

        @pl.when(i + _DEPTH < _NBLK)
        def _refill():
            _a_copy(a_hbm, a_buf, sems, i + _DEPTH, slot).start()

        return carry

    jax.lax.fori_loop(0, _NBLK, body, 0)


def kernel(adjacency, features_matrix, weights, bias):
    bias2d = bias.reshape(1, _D_OUT)
    return pl.pallas_call(
        _gcn_kernel,
        in_specs=[
            pl.BlockSpec(memory_space=pltpu.MemorySpace.VMEM),
            pl.BlockSpec(memory_space=pltpu.MemorySpace.VMEM),
            pl.BlockSpec(memory_space=pltpu.MemorySpace.VMEM),
            pl.BlockSpec(memory_space=pltpu.MemorySpace.HBM),
        ],
        out_specs=pl.BlockSpec(memory_space=pltpu.MemorySpace.VMEM),
        out_shape=jax.ShapeDtypeStruct((_N, _D_OUT), jnp.float32),
        scratch_shapes=[
            pltpu.VMEM((_N, _D_IN), jnp.float32),
            pltpu.VMEM((_DEPTH, _BM, _N), jnp.float32),
            pltpu.SemaphoreType.DMA((_DEPTH,)),
        ],
    )(features_matrix, weights, bias2d, adjacency)
